# Initial kernel scaffold; baseline (speedup 1.0000x reference)
import jax
import jax.numpy as jnp
from jax.experimental import pallas as pl


def kernel(data, merge, structure, w_conv, b_conv, w_t, b_t):
    raise NotImplementedError("write your pallas kernel here")


# speedup vs baseline: 9.9056x; 9.9056x over previous
import jax
import jax.numpy as jnp
from jax import lax
from jax.experimental import pallas as pl
from jax.experimental.pallas import tpu as pltpu
from jax.experimental.pallas import tpu_sc as plsc

N = 10000
E = 320000
F = 192
FH = F // 2
NC, NS = 2, 16
EPT = E // NS
BLK = 80
NBLK = EPT // BLK
RPT = 640
NPAD = NS * RPT
R = 1000


def _conv_body(x_ref, wc_ref, b_ref, o_ref):
    x = x_ref[...]
    outs = []
    for g in range(3):
        outs.append(jnp.dot(x[:, g * 64:(g + 1) * 64], wc_ref[g],
                            preferred_element_type=jnp.float32))
    res = jnp.concatenate(outs, axis=1) + b_ref[...]
    o_ref[0] = res[:, :FH]
    o_ref[1] = res[:, FH:]


_conv_call = pl.pallas_call(
    _conv_body,
    grid=(N // R,),
    in_specs=[pl.BlockSpec((R, F), lambda i: (i, 0)),
              pl.BlockSpec((3, 64, 64), lambda i: (0, 0, 0)),
              pl.BlockSpec((1, F), lambda i: (0, 0))],
    out_specs=pl.BlockSpec((2, R, FH), lambda i: (0, i, 0)),
    out_shape=jax.ShapeDtypeStruct((2, N, FH), jnp.float32),
)


def _combine_body(olo_ref, ohi_ref, mg_ref, wt_ref, bt_ref, nbr_ref,
                  dg_ref, o_ref):
    t = jnp.dot(mg_ref[...], wt_ref[...],
                preferred_element_type=jnp.float32) + bt_ref[...]
    d = dg_ref[...]
    out = jnp.concatenate([olo_ref[0], ohi_ref[0]], axis=1)
    lap = jnp.where(d > 0.0, out - nbr_ref[...] / jnp.maximum(d, 1.0), 0.0)
    o_ref[...] = jnp.maximum(lap + t, 0.0)


_combine_call = pl.pallas_call(
    _combine_body,
    grid=(N // R,),
    in_specs=[pl.BlockSpec((1, R, FH), lambda i: (0, i, 0)),
              pl.BlockSpec((1, R, FH), lambda i: (1, i, 0)),
              pl.BlockSpec((R, 64), lambda i: (i, 0)),
              pl.BlockSpec((64, F), lambda i: (0, 0)),
              pl.BlockSpec((1, F), lambda i: (0, 0)),
              pl.BlockSpec((R, F), lambda i: (i, 0)),
              pl.BlockSpec((R, 1), lambda i: (i, 0))],
    out_specs=pl.BlockSpec((R, F), lambda i: (i, 0)),
    out_shape=jax.ShapeDtypeStruct((N, F), jnp.float32),
)


def _sc_body(out_hbm, src_hbm, dst_hbm, zacc_hbm, zdeg_hbm, ones_hbm,
             nbr_out, deg_out, acc_sh, deg_sh, src_v, dst_v, rows, ones_v):
    cid = lax.axis_index("c")
    sid = lax.axis_index("s")
    wid = cid * NS + sid

    pltpu.sync_copy(zacc_hbm, acc_sh.at[pl.ds(sid * RPT, RPT)])
    pltpu.sync_copy(zdeg_hbm, deg_sh.at[pl.ds(sid * RPT, RPT)])
    pltpu.sync_copy(src_hbm.at[wid], src_v)
    pltpu.sync_copy(dst_hbm.at[sid], dst_v)
    pltpu.sync_copy(ones_hbm, ones_v)
    plsc.subcore_barrier()

    @pl.when(cid == 0)
    def _core0():
        def body0(j, carry):
            pltpu.sync_copy(out_hbm.at[src_v.at[j]], rows)
            pltpu.sync_copy(rows, acc_sh.at[dst_v.at[j]], add=True)
            pltpu.sync_copy(ones_v, deg_sh.at[dst_v.at[j]], add=True)
            return carry
        lax.fori_loop(0, NBLK, body0, 0)

    @pl.when(cid == 1)
    def _core1():
        def body1(j, carry):
            pltpu.sync_copy(out_hbm.at[src_v.at[j]], rows)
            pltpu.sync_copy(rows, acc_sh.at[dst_v.at[j]], add=True)
            return carry
        lax.fori_loop(0, NBLK, body1, 0)

    plsc.subcore_barrier()
    pltpu.sync_copy(acc_sh.at[pl.ds(sid * RPT, RPT)],
                    nbr_out.at[pl.ds(sid * RPT, RPT), pl.ds(cid * FH, FH)])

    @pl.when(cid == 0)
    def _deg_out():
        pltpu.sync_copy(deg_sh.at[pl.ds(sid * RPT, RPT)],
                        deg_out.at[pl.ds(sid * RPT, RPT)])


_scatter_call = pl.kernel(
    _sc_body,
    out_type=[jax.ShapeDtypeStruct((NPAD, F), jnp.float32),
              jax.ShapeDtypeStruct((NPAD,), jnp.float32)],
    mesh=plsc.VectorSubcoreMesh(core_axis_name="c", subcore_axis_name="s"),
    compiler_params=pltpu.CompilerParams(use_tc_tiling_on_sc=False),
    scratch_types=[
        pltpu.VMEM_SHARED((NPAD, FH), jnp.float32),
        pltpu.VMEM_SHARED((NPAD,), jnp.float32),
        pltpu.VMEM((NBLK, BLK), jnp.int32),
        pltpu.VMEM((NBLK, BLK), jnp.int32),
        pltpu.VMEM((BLK, FH), jnp.float32),
        pltpu.VMEM((BLK,), jnp.float32),
    ],
)


def kernel(data, merge, structure, w_conv, b_conv, w_t, b_t):
    wc_t = w_conv.transpose(0, 2, 1)
    b_flat = b_conv.reshape(1, F)
    out3 = _conv_call(data, wc_t, b_flat)
    out2 = out3.reshape(2 * N, FH)

    s0 = structure[0].reshape(NS, NBLK, BLK)
    src_aug = jnp.concatenate([s0, s0 + N], axis=0)
    dst_r = structure[1].reshape(NS, NBLK, BLK)
    zacc = jnp.zeros((RPT, FH), jnp.float32)
    zdeg = jnp.zeros((RPT,), jnp.float32)
    ones = jnp.ones((BLK,), jnp.float32)
    nbr, deg = _scatter_call(out2, src_aug, dst_r, zacc, zdeg, ones)

    dg = deg[:N].reshape(N, 1)
    return _combine_call(out3, out3, merge, w_t, b_t.reshape(1, F),
                         nbr, dg)


# speedup vs baseline: 14.9251x; 1.5067x over previous
import jax
import jax.numpy as jnp
from jax import lax
from jax.experimental import pallas as pl
from jax.experimental.pallas import tpu as pltpu
from jax.experimental.pallas import tpu_sc as plsc

N = 10000
E = 320000
F = 192
FH = F // 2
NC, NS = 2, 16
EPT = E // NS
BLK = 80
NBLK = EPT // BLK
RPT = 640
NPAD = NS * RPT
R = 1000


def _conv_body(x_ref, wc_ref, b_ref, o_ref):
    x = x_ref[...]
    outs = []
    for g in range(3):
        outs.append(jnp.dot(x[:, g * 64:(g + 1) * 64], wc_ref[g],
                            preferred_element_type=jnp.float32))
    res = jnp.concatenate(outs, axis=1) + b_ref[...]
    o_ref[0] = res[:, :FH]
    o_ref[1] = res[:, FH:]


_conv_call = pl.pallas_call(
    _conv_body,
    grid=(N // R,),
    in_specs=[pl.BlockSpec((R, F), lambda i: (i, 0)),
              pl.BlockSpec((3, 64, 64), lambda i: (0, 0, 0)),
              pl.BlockSpec((1, F), lambda i: (0, 0))],
    out_specs=pl.BlockSpec((2, R, FH), lambda i: (0, i, 0)),
    out_shape=jax.ShapeDtypeStruct((2, N, FH), jnp.float32),
)


def _combine_body(olo_ref, ohi_ref, mg_ref, wt_ref, bt_ref, nbr_ref,
                  dg_ref, o_ref):
    t = jnp.dot(mg_ref[...], wt_ref[...],
                preferred_element_type=jnp.float32) + bt_ref[...]
    d = dg_ref[...]
    out = jnp.concatenate([olo_ref[0], ohi_ref[0]], axis=1)
    lap = jnp.where(d > 0.0, out - nbr_ref[...] / jnp.maximum(d, 1.0), 0.0)
    o_ref[...] = jnp.maximum(lap + t, 0.0)


_combine_call = pl.pallas_call(
    _combine_body,
    grid=(N // R,),
    in_specs=[pl.BlockSpec((1, R, FH), lambda i: (0, i, 0)),
              pl.BlockSpec((1, R, FH), lambda i: (1, i, 0)),
              pl.BlockSpec((R, 64), lambda i: (i, 0)),
              pl.BlockSpec((64, F), lambda i: (0, 0)),
              pl.BlockSpec((1, F), lambda i: (0, 0)),
              pl.BlockSpec((R, F), lambda i: (i, 0)),
              pl.BlockSpec((R, 1), lambda i: (i, 0))],
    out_specs=pl.BlockSpec((R, F), lambda i: (i, 0)),
    out_shape=jax.ShapeDtypeStruct((N, F), jnp.float32),
)


def _sc_body(out_hbm, src_hbm, dst_hbm, zacc_hbm, zdeg_hbm, ones_hbm,
             nbr_out, deg_out, acc_sh, deg_sh, src_v, dst_v, rows0, rows1,
             ones_v, sem0, sem1):
    cid = lax.axis_index("c")
    sid = lax.axis_index("s")
    wid = cid * NS + sid

    pltpu.sync_copy(zacc_hbm, acc_sh.at[pl.ds(sid * RPT, RPT)])
    pltpu.sync_copy(zdeg_hbm, deg_sh.at[pl.ds(sid * RPT, RPT)])
    pltpu.sync_copy(src_hbm.at[wid], src_v)
    pltpu.sync_copy(dst_hbm.at[sid], dst_v)
    pltpu.sync_copy(ones_hbm, ones_v)
    plsc.subcore_barrier()

    pltpu.async_copy(out_hbm.at[src_v.at[0]], rows0, sem0)
    pltpu.async_copy(out_hbm.at[src_v.at[1]], rows1, sem1)

    def half(jj, j, rows, sem):
        pltpu.make_async_copy(out_hbm.at[src_v.at[j]], rows, sem).wait()
        pltpu.sync_copy(rows, acc_sh.at[dst_v.at[j]], add=True)

        @pl.when(cid == 0)
        def _deg():
            pltpu.sync_copy(ones_v, deg_sh.at[dst_v.at[j]], add=True)

        @pl.when(jj < (NBLK // 2) - 1)
        def _next():
            jn = jnp.minimum(j + 2, NBLK - 1)
            pltpu.async_copy(out_hbm.at[src_v.at[jn]], rows, sem)

    def pair(jj, carry):
        j0 = 2 * jj
        half(jj, j0, rows0, sem0)
        half(jj, j0 + 1, rows1, sem1)
        return carry

    lax.fori_loop(0, NBLK // 2, pair, 0)

    plsc.subcore_barrier()
    pltpu.sync_copy(acc_sh.at[pl.ds(sid * RPT, RPT)],
                    nbr_out.at[pl.ds(sid * RPT, RPT), pl.ds(cid * FH, FH)])

    @pl.when(cid == 0)
    def _deg_out():
        pltpu.sync_copy(deg_sh.at[pl.ds(sid * RPT, RPT)],
                        deg_out.at[pl.ds(sid * RPT, RPT)])


_scatter_call = pl.kernel(
    _sc_body,
    out_type=[jax.ShapeDtypeStruct((NPAD, F), jnp.float32),
              jax.ShapeDtypeStruct((NPAD,), jnp.float32)],
    mesh=plsc.VectorSubcoreMesh(core_axis_name="c", subcore_axis_name="s"),
    compiler_params=pltpu.CompilerParams(use_tc_tiling_on_sc=False),
    scratch_types=[
        pltpu.VMEM_SHARED((NPAD, FH), jnp.float32),
        pltpu.VMEM_SHARED((NPAD,), jnp.float32),
        pltpu.VMEM((NBLK, BLK), jnp.int32),
        pltpu.VMEM((NBLK, BLK), jnp.int32),
        pltpu.VMEM((BLK, FH), jnp.float32),
        pltpu.VMEM((BLK, FH), jnp.float32),
        pltpu.VMEM((BLK,), jnp.float32),
        pltpu.SemaphoreType.DMA,
        pltpu.SemaphoreType.DMA,
    ],
)


def kernel(data, merge, structure, w_conv, b_conv, w_t, b_t):
    wc_t = w_conv.transpose(0, 2, 1)
    b_flat = b_conv.reshape(1, F)
    out3 = _conv_call(data, wc_t, b_flat)
    out2 = out3.reshape(2 * N, FH)

    s0 = structure[0].reshape(NS, NBLK, BLK)
    src_aug = jnp.concatenate([s0, s0 + N], axis=0)
    dst_r = structure[1].reshape(NS, NBLK, BLK)
    zacc = jnp.zeros((RPT, FH), jnp.float32)
    zdeg = jnp.zeros((RPT,), jnp.float32)
    ones = jnp.ones((BLK,), jnp.float32)
    nbr, deg = _scatter_call(out2, src_aug, dst_r, zacc, zdeg, ones)

    dg = deg[:N].reshape(N, 1)
    return _combine_call(out3, out3, merge, w_t, b_t.reshape(1, F),
                         nbr, dg)


# speedup vs baseline: 16.8162x; 1.1267x over previous
import jax
import jax.numpy as jnp
from jax import lax
from jax.experimental import pallas as pl
from jax.experimental.pallas import tpu as pltpu
from jax.experimental.pallas import tpu_sc as plsc

N = 10000
E = 320000
F = 192
FH = F // 2
NC, NS = 2, 16
EPT = E // NS
BLK = 80
NBLK = EPT // BLK
RPT = 640
NPAD = NS * RPT
R = 1000


def _conv_body(x_ref, wc_ref, b_ref, o_ref):
    x = x_ref[...]
    outs = []
    for g in range(3):
        outs.append(jnp.dot(x[:, g * 64:(g + 1) * 64], wc_ref[g],
                            preferred_element_type=jnp.float32))
    res = jnp.concatenate(outs, axis=1) + b_ref[...]
    o_ref[0] = res[:, :FH]
    o_ref[1] = res[:, FH:]


_conv_call = pl.pallas_call(
    _conv_body,
    grid=(N // R,),
    in_specs=[pl.BlockSpec((R, F), lambda i: (i, 0)),
              pl.BlockSpec((3, 64, 64), lambda i: (0, 0, 0)),
              pl.BlockSpec((1, F), lambda i: (0, 0))],
    out_specs=pl.BlockSpec((2, R, FH), lambda i: (0, i, 0)),
    out_shape=jax.ShapeDtypeStruct((2, N, FH), jnp.float32),
)


def _combine_body(olo_ref, ohi_ref, mg_ref, wt_ref, bt_ref, nbr_ref,
                  dg_ref, o_ref):
    t = jnp.dot(mg_ref[...], wt_ref[...],
                preferred_element_type=jnp.float32) + bt_ref[...]
    d = dg_ref[...]
    out = jnp.concatenate([olo_ref[0], ohi_ref[0]], axis=1)
    lap = jnp.where(d > 0.0, out - nbr_ref[...] / jnp.maximum(d, 1.0), 0.0)
    o_ref[...] = jnp.maximum(lap + t, 0.0)


_combine_call = pl.pallas_call(
    _combine_body,
    grid=(N // R,),
    in_specs=[pl.BlockSpec((1, R, FH), lambda i: (0, i, 0)),
              pl.BlockSpec((1, R, FH), lambda i: (1, i, 0)),
              pl.BlockSpec((R, 64), lambda i: (i, 0)),
              pl.BlockSpec((64, F), lambda i: (0, 0)),
              pl.BlockSpec((1, F), lambda i: (0, 0)),
              pl.BlockSpec((R, F), lambda i: (i, 0)),
              pl.BlockSpec((R, 1), lambda i: (i, 0))],
    out_specs=pl.BlockSpec((R, F), lambda i: (i, 0)),
    out_shape=jax.ShapeDtypeStruct((N, F), jnp.float32),
)


NBUF = 3


def _sc_body(out_hbm, src_hbm, dst_hbm, zacc_hbm, zdeg_hbm, ones_hbm,
             nbr_out, deg_out, acc_sh, deg_sh, src_v, dst_v, ring,
             ones_v, gsem, ssem):
    cid = lax.axis_index("c")
    sid = lax.axis_index("s")
    wid = cid * NS + sid

    pltpu.sync_copy(zacc_hbm, acc_sh.at[pl.ds(sid * RPT, RPT)])
    pltpu.sync_copy(zdeg_hbm, deg_sh.at[pl.ds(sid * RPT, RPT)])
    pltpu.sync_copy(src_hbm.at[wid], src_v)
    pltpu.sync_copy(dst_hbm.at[sid], dst_v)
    pltpu.sync_copy(ones_hbm, ones_v)
    plsc.subcore_barrier()

    pltpu.async_copy(out_hbm.at[src_v.at[0]], ring.at[0], gsem.at[0])
    pltpu.async_copy(out_hbm.at[src_v.at[1]], ring.at[1], gsem.at[1])

    def body(j, carry):
        b = lax.rem(j, NBUF)
        pltpu.make_async_copy(out_hbm.at[src_v.at[j]], ring.at[b],
                              gsem.at[b]).wait()
        pltpu.async_copy(ring.at[b], acc_sh.at[dst_v.at[j]], ssem.at[b],
                         add=True)

        do_deg = jnp.logical_or(
            jnp.logical_and(cid == 0, j < NBLK // 2),
            jnp.logical_and(cid == 1, j >= NBLK // 2))

        @pl.when(do_deg)
        def _deg():
            pltpu.sync_copy(ones_v, deg_sh.at[dst_v.at[j]], add=True)

        jn = j + 2

        @pl.when(jn < NBLK)
        def _next():
            bn = lax.rem(jn, NBUF)

            @pl.when(jn >= NBUF)
            def _reuse():
                pltpu.make_async_copy(ring.at[bn], acc_sh.at[dst_v.at[j]],
                                      ssem.at[bn]).wait()

            pltpu.async_copy(out_hbm.at[src_v.at[jn]], ring.at[bn],
                             gsem.at[bn])

        return carry

    lax.fori_loop(0, NBLK, body, 0)
    for bt in range(NBUF):
        pltpu.make_async_copy(ring.at[bt], acc_sh.at[dst_v.at[0]],
                              ssem.at[bt]).wait()

    plsc.subcore_barrier()
    pltpu.sync_copy(acc_sh.at[pl.ds(sid * RPT, RPT)],
                    nbr_out.at[pl.ds(sid * RPT, RPT), pl.ds(cid * FH, FH)])
    pltpu.sync_copy(deg_sh.at[pl.ds(sid * RPT, RPT)],
                    deg_out.at[pl.ds(cid * NPAD + sid * RPT, RPT)])


_scatter_call = pl.kernel(
    _sc_body,
    out_type=[jax.ShapeDtypeStruct((NPAD, F), jnp.float32),
              jax.ShapeDtypeStruct((2 * NPAD,), jnp.float32)],
    mesh=plsc.VectorSubcoreMesh(core_axis_name="c", subcore_axis_name="s"),
    compiler_params=pltpu.CompilerParams(use_tc_tiling_on_sc=False),
    scratch_types=[
        pltpu.VMEM_SHARED((NPAD, FH), jnp.float32),
        pltpu.VMEM_SHARED((NPAD,), jnp.float32),
        pltpu.VMEM((NBLK, BLK), jnp.int32),
        pltpu.VMEM((NBLK, BLK), jnp.int32),
        pltpu.VMEM((NBUF, BLK, FH), jnp.float32),
        pltpu.VMEM((BLK,), jnp.float32),
        pltpu.SemaphoreType.DMA((NBUF,)),
        pltpu.SemaphoreType.DMA((NBUF,)),
    ],
)


def kernel(data, merge, structure, w_conv, b_conv, w_t, b_t):
    wc_t = w_conv.transpose(0, 2, 1)
    b_flat = b_conv.reshape(1, F)
    out3 = _conv_call(data, wc_t, b_flat)
    out2 = out3.reshape(2 * N, FH)

    s0 = structure[0].reshape(NS, NBLK, BLK)
    src_aug = jnp.concatenate([s0, s0 + N], axis=0)
    dst_r = structure[1].reshape(NS, NBLK, BLK)
    zacc = jnp.zeros((RPT, FH), jnp.float32)
    zdeg = jnp.zeros((RPT,), jnp.float32)
    ones = jnp.ones((BLK,), jnp.float32)
    nbr, deg = _scatter_call(out2, src_aug, dst_r, zacc, zdeg, ones)

    dg = (deg[:N] + deg[NPAD:NPAD + N]).reshape(N, 1)
    return _combine_call(out3, out3, merge, w_t, b_t.reshape(1, F),
                         nbr, dg)


# speedup vs baseline: 17.3012x; 1.0288x over previous
import jax
import jax.numpy as jnp
from jax import lax
from jax.experimental import pallas as pl
from jax.experimental.pallas import tpu as pltpu
from jax.experimental.pallas import tpu_sc as plsc

N = 10000
E = 320000
F = 192
FH = F // 2
NC, NS = 2, 16
EPT = E // NS
BLK = 80
NBLK = EPT // BLK
RPT = 640
NPAD = NS * RPT
R = 1000


def _conv_body(x_ref, wc_ref, b_ref, o_ref):
    x = x_ref[...]
    outs = []
    for g in range(3):
        outs.append(jnp.dot(x[:, g * 64:(g + 1) * 64], wc_ref[g],
                            preferred_element_type=jnp.float32))
    res = jnp.concatenate(outs, axis=1) + b_ref[...]
    o_ref[0] = res[:, :FH]
    o_ref[1] = res[:, FH:]


_conv_call = pl.pallas_call(
    _conv_body,
    grid=(N // R,),
    in_specs=[pl.BlockSpec((R, F), lambda i: (i, 0)),
              pl.BlockSpec((3, 64, 64), lambda i: (0, 0, 0)),
              pl.BlockSpec((1, F), lambda i: (0, 0))],
    out_specs=pl.BlockSpec((2, R, FH), lambda i: (0, i, 0)),
    out_shape=jax.ShapeDtypeStruct((2, N, FH), jnp.float32),
)


def _combine_body(olo_ref, ohi_ref, mg_ref, wt_ref, bt_ref, nbr_ref,
                  dg_ref, o_ref):
    t = jnp.dot(mg_ref[...], wt_ref[...],
                preferred_element_type=jnp.float32) + bt_ref[...]
    d = dg_ref[...]
    out = jnp.concatenate([olo_ref[0], ohi_ref[0]], axis=1)
    lap = jnp.where(d > 0.0, out - nbr_ref[...] / jnp.maximum(d, 1.0), 0.0)
    o_ref[...] = jnp.maximum(lap + t, 0.0)


_combine_call = pl.pallas_call(
    _combine_body,
    grid=(N // R,),
    in_specs=[pl.BlockSpec((1, R, FH), lambda i: (0, i, 0)),
              pl.BlockSpec((1, R, FH), lambda i: (1, i, 0)),
              pl.BlockSpec((R, 64), lambda i: (i, 0)),
              pl.BlockSpec((64, F), lambda i: (0, 0)),
              pl.BlockSpec((1, F), lambda i: (0, 0)),
              pl.BlockSpec((R, F), lambda i: (i, 0)),
              pl.BlockSpec((R, 1), lambda i: (i, 0))],
    out_specs=pl.BlockSpec((R, F), lambda i: (i, 0)),
    out_shape=jax.ShapeDtypeStruct((N, F), jnp.float32),
)


NBUF = 3


def _sc_body(out_hbm, src_hbm, dst_hbm, zacc_hbm, zdeg_hbm, ones_hbm,
             nbr_out, deg_out, acc_sh, deg_sh, src_v, dst_v, ring,
             ones_v, gsem, ssem):
    cid = lax.axis_index("c")
    sid = lax.axis_index("s")
    out_half = out_hbm.at[cid]

    pltpu.sync_copy(zacc_hbm, acc_sh.at[pl.ds(sid * RPT, RPT)])
    pltpu.sync_copy(zdeg_hbm, deg_sh.at[pl.ds(sid * RPT, RPT)])
    pltpu.sync_copy(src_hbm.at[sid], src_v)
    pltpu.sync_copy(dst_hbm.at[sid], dst_v)
    pltpu.sync_copy(ones_hbm, ones_v)
    plsc.subcore_barrier()

    pltpu.async_copy(out_half.at[src_v.at[0]], ring.at[0], gsem.at[0])
    pltpu.async_copy(out_half.at[src_v.at[1]], ring.at[1], gsem.at[1])

    def body(j, carry):
        b = lax.rem(j, NBUF)
        pltpu.make_async_copy(out_half.at[src_v.at[j]], ring.at[b],
                              gsem.at[b]).wait()
        pltpu.async_copy(ring.at[b], acc_sh.at[dst_v.at[j]], ssem.at[b],
                         add=True)

        do_deg = jnp.logical_or(
            jnp.logical_and(cid == 0, j < NBLK // 2),
            jnp.logical_and(cid == 1, j >= NBLK // 2))

        @pl.when(do_deg)
        def _deg():
            pltpu.sync_copy(ones_v, deg_sh.at[dst_v.at[j]], add=True)

        jn = j + 2

        @pl.when(jn < NBLK)
        def _next():
            bn = lax.rem(jn, NBUF)

            @pl.when(jn >= NBUF)
            def _reuse():
                pltpu.make_async_copy(ring.at[bn], acc_sh.at[dst_v.at[j]],
                                      ssem.at[bn]).wait()

            pltpu.async_copy(out_half.at[src_v.at[jn]], ring.at[bn],
                             gsem.at[bn])

        return carry

    lax.fori_loop(0, NBLK, body, 0)
    for bt in range(NBUF):
        pltpu.make_async_copy(ring.at[bt], acc_sh.at[dst_v.at[0]],
                              ssem.at[bt]).wait()

    plsc.subcore_barrier()
    pltpu.sync_copy(acc_sh.at[pl.ds(sid * RPT, RPT)],
                    nbr_out.at[pl.ds(sid * RPT, RPT), pl.ds(cid * FH, FH)])
    pltpu.sync_copy(deg_sh.at[pl.ds(sid * RPT, RPT)],
                    deg_out.at[pl.ds(cid * NPAD + sid * RPT, RPT)])


_scatter_call = pl.kernel(
    _sc_body,
    out_type=[jax.ShapeDtypeStruct((NPAD, F), jnp.float32),
              jax.ShapeDtypeStruct((2 * NPAD,), jnp.float32)],
    mesh=plsc.VectorSubcoreMesh(core_axis_name="c", subcore_axis_name="s"),
    compiler_params=pltpu.CompilerParams(use_tc_tiling_on_sc=False),
    scratch_types=[
        pltpu.VMEM_SHARED((NPAD, FH), jnp.float32),
        pltpu.VMEM_SHARED((NPAD,), jnp.float32),
        pltpu.VMEM((NBLK, BLK), jnp.int32),
        pltpu.VMEM((NBLK, BLK), jnp.int32),
        pltpu.VMEM((NBUF, BLK, FH), jnp.float32),
        pltpu.VMEM((BLK,), jnp.float32),
        pltpu.SemaphoreType.DMA((NBUF,)),
        pltpu.SemaphoreType.DMA((NBUF,)),
    ],
)


def kernel(data, merge, structure, w_conv, b_conv, w_t, b_t):
    wc_t = w_conv.transpose(0, 2, 1)
    b_flat = b_conv.reshape(1, F)
    out3 = _conv_call(data, wc_t, b_flat)

    src_r = structure[0].reshape(NS, NBLK, BLK)
    dst_r = structure[1].reshape(NS, NBLK, BLK)
    zacc = jnp.zeros((RPT, FH), jnp.float32)
    zdeg = jnp.zeros((RPT,), jnp.float32)
    ones = jnp.ones((BLK,), jnp.float32)
    nbr, deg = _scatter_call(out3, src_r, dst_r, zacc, zdeg, ones)

    dg = (deg[:N] + deg[NPAD:NPAD + N]).reshape(N, 1)
    return _combine_call(out3, out3, merge, w_t, b_t.reshape(1, F),
                         nbr, dg)
